# Initial kernel scaffold; baseline (speedup 1.0000x reference)
import jax
import jax.numpy as jnp
from jax.experimental import pallas as pl


def kernel(x, edge_index, edge_attr, node_w, node_b, c0_init_w, c0_root_w, c0_bias, bn0_gamma, bn0_beta, c1_init_w, c1_root_w, c1_bias, bn1_gamma, bn1_beta, c2_init_w, c2_root_w, c2_bias, bn2_gamma, bn2_beta, el1_w, el1_b, el2_w, el2_b, mlp_w1, mlp_b1, mlp_w2, mlp_b2):
    raise NotImplementedError("write your pallas kernel here")


# speedup vs baseline: 22.8205x; 22.8205x over previous
import jax
import jax.numpy as jnp
from jax import lax
from jax.experimental import pallas as pl
from jax.experimental.pallas import tpu as pltpu
from jax.experimental.pallas import tpu_sc as plsc

N = 10000
E = 320000
H = 32
NC = 2
NS = 16
NW = NC * NS
K = 500
CHW = E // (NW * K)
EB = 12800
DEGW = 8
NP = N // 4
F32 = jnp.float32

_mesh = plsc.VectorSubcoreMesh(core_axis_name="c", subcore_axis_name="s")
_SC_PARAMS = pltpu.CompilerParams(use_tc_tiling_on_sc=False)


def _worker_id():
  return lax.axis_index("c") * NS + lax.axis_index("s")


def _deg_body(col2_hbm, ones_hbm, zeros_hbm, out_hbm, colb, onesb, deg_sh):
  c = lax.axis_index("c")
  s = lax.axis_index("s")
  w = _worker_id()
  pltpu.sync_copy(col2_hbm.at[w], colb)
  pltpu.sync_copy(ones_hbm, onesb)
  _zero_stripes(zeros_hbm, deg_sh, s)
  plsc.subcore_barrier()

  @pl.loop(0, CHW)
  def _(j):
    pltpu.sync_copy(onesb, deg_sh.at[colb.at[j]], add=True)

  plsc.subcore_barrier()
  _read_stripes(deg_sh, out_hbm, c, s)


def _sc_degree(col2, ones_k, zeros_n):
  kfn = pl.kernel(
      _deg_body,
      out_type=jax.ShapeDtypeStruct((NC, N, DEGW), F32),
      mesh=_mesh,
      compiler_params=_SC_PARAMS,
      scratch_types=[
          pltpu.VMEM((CHW, K), jnp.int32),
          pltpu.VMEM((K, DEGW), F32),
          pltpu.VMEM_SHARED((N, DEGW), F32),
      ],
  )
  return kfn(col2, ones_k, zeros_n)


def _bd4(w):
  z = jnp.zeros(w.shape, F32)
  rows = [jnp.concatenate([z] * i + [w] + [z] * (3 - i), axis=1)
          for i in range(4)]
  return jnp.concatenate(rows, axis=0)


def _tile4(v):
  return jnp.concatenate([v] * 4, axis=1)


def _zero_stripes(zeros_hbm, table_sh, s):
  @pl.when(s < NS - 1)
  def _():
    pltpu.sync_copy(zeros_hbm, table_sh.at[pl.ds(s * 640, 640)])

  @pl.when(s == NS - 1)
  def _():
    pltpu.sync_copy(zeros_hbm.at[pl.ds(0, 400)],
                    table_sh.at[pl.ds(9600, 400)])


def _read_stripes(table_sh, out_hbm, c, s):
  @pl.when(s < NS - 1)
  def _():
    pltpu.sync_copy(table_sh.at[pl.ds(s * 640, 640)],
                    out_hbm.at[c, pl.ds(s * 640, 640)])

  @pl.when(s == NS - 1)
  def _():
    pltpu.sync_copy(table_sh.at[pl.ds(9600, 400)],
                    out_hbm.at[c, pl.ds(9600, 400)])


def _agg_body(g_hbm, row2_hbm, col2_hbm, zeros_hbm, out_hbm,
              rowb, colb, d0, d1, agg_sh, gs0, gs1, ss0, ss1):
  c = lax.axis_index("c")
  s = lax.axis_index("s")
  w = _worker_id()
  pltpu.sync_copy(row2_hbm.at[w], rowb)
  pltpu.sync_copy(col2_hbm.at[w], colb)
  _zero_stripes(zeros_hbm, agg_sh, s)
  plsc.subcore_barrier()

  pltpu.async_copy(g_hbm.at[rowb.at[0]], d0, gs0)
  pltpu.async_copy(g_hbm.at[rowb.at[1]], d1, gs1)

  @pl.loop(0, CHW, step=2)
  def _(j):
    pltpu.make_async_copy(g_hbm.at[rowb.at[j]], d0, gs0).wait()
    pltpu.async_copy(d0, agg_sh.at[colb.at[j]], ss0, add=True)
    pltpu.make_async_copy(g_hbm.at[rowb.at[j + 1]], d1, gs1).wait()
    pltpu.async_copy(d1, agg_sh.at[colb.at[j + 1]], ss1, add=True)

    @pl.when(j + 2 < CHW)
    def _():
      pltpu.make_async_copy(d0, agg_sh.at[colb.at[j]], ss0).wait()
      pltpu.async_copy(g_hbm.at[rowb.at[j + 2]], d0, gs0)
      pltpu.make_async_copy(d1, agg_sh.at[colb.at[j + 1]], ss1).wait()
      pltpu.async_copy(g_hbm.at[rowb.at[j + 3]], d1, gs1)

  pltpu.make_async_copy(d0, agg_sh.at[colb.at[CHW - 2]], ss0).wait()
  pltpu.make_async_copy(d1, agg_sh.at[colb.at[CHW - 1]], ss1).wait()
  plsc.subcore_barrier()
  _read_stripes(agg_sh, out_hbm, c, s)


def _sc_gather_scatter(g, row2, col2, zeros_stripe):
  kfn = pl.kernel(
      _agg_body,
      out_type=jax.ShapeDtypeStruct((NC, N, H), F32),
      mesh=_mesh,
      compiler_params=_SC_PARAMS,
      scratch_types=[
          pltpu.VMEM((CHW, K), jnp.int32),
          pltpu.VMEM((CHW, K), jnp.int32),
          pltpu.VMEM((K, H), F32),
          pltpu.VMEM((K, H), F32),
          pltpu.VMEM_SHARED((N, H), F32),
          pltpu.SemaphoreType.DMA,
          pltpu.SemaphoreType.DMA,
          pltpu.SemaphoreType.DMA,
          pltpu.SemaphoreType.DMA,
      ],
  )
  return kfn(g, row2, col2, zeros_stripe)


def _pair_body(a_hbm, b_hbm, row2_hbm, col2_hbm, pa_hbm, pb_hbm,
               rowb, colb, a0, a1, b0, b1,
               gsa0, gsa1, gsb0, gsb1, wsa0, wsa1, wsb0, wsb1):
  w = _worker_id()
  pltpu.sync_copy(row2_hbm.at[w], rowb)
  pltpu.sync_copy(col2_hbm.at[w], colb)
  base0 = w * CHW * K

  pltpu.async_copy(a_hbm.at[rowb.at[0]], a0, gsa0)
  pltpu.async_copy(b_hbm.at[colb.at[0]], b0, gsb0)
  pltpu.async_copy(a_hbm.at[rowb.at[1]], a1, gsa1)
  pltpu.async_copy(b_hbm.at[colb.at[1]], b1, gsb1)

  @pl.loop(0, CHW, step=2)
  def _(j):
    pltpu.make_async_copy(a_hbm.at[rowb.at[j]], a0, gsa0).wait()
    pltpu.async_copy(a0, pa_hbm.at[pl.ds(base0 + j * K, K)], wsa0)
    pltpu.make_async_copy(b_hbm.at[colb.at[j]], b0, gsb0).wait()
    pltpu.async_copy(b0, pb_hbm.at[pl.ds(base0 + j * K, K)], wsb0)
    pltpu.make_async_copy(a_hbm.at[rowb.at[j + 1]], a1, gsa1).wait()
    pltpu.async_copy(a1, pa_hbm.at[pl.ds(base0 + (j + 1) * K, K)], wsa1)
    pltpu.make_async_copy(b_hbm.at[colb.at[j + 1]], b1, gsb1).wait()
    pltpu.async_copy(b1, pb_hbm.at[pl.ds(base0 + (j + 1) * K, K)], wsb1)

    @pl.when(j + 2 < CHW)
    def _():
      pltpu.make_async_copy(a0, pa_hbm.at[pl.ds(base0 + j * K, K)],
                            wsa0).wait()
      pltpu.async_copy(a_hbm.at[rowb.at[j + 2]], a0, gsa0)
      pltpu.make_async_copy(b0, pb_hbm.at[pl.ds(base0 + j * K, K)],
                            wsb0).wait()
      pltpu.async_copy(b_hbm.at[colb.at[j + 2]], b0, gsb0)
      pltpu.make_async_copy(a1, pa_hbm.at[pl.ds(base0 + (j + 1) * K, K)],
                            wsa1).wait()
      pltpu.async_copy(a_hbm.at[rowb.at[j + 3]], a1, gsa1)
      pltpu.make_async_copy(b1, pb_hbm.at[pl.ds(base0 + (j + 1) * K, K)],
                            wsb1).wait()
      pltpu.async_copy(b_hbm.at[colb.at[j + 3]], b1, gsb1)

  pltpu.make_async_copy(a0, pa_hbm.at[pl.ds(base0 + (CHW - 2) * K, K)],
                        wsa0).wait()
  pltpu.make_async_copy(b0, pb_hbm.at[pl.ds(base0 + (CHW - 2) * K, K)],
                        wsb0).wait()
  pltpu.make_async_copy(a1, pa_hbm.at[pl.ds(base0 + (CHW - 1) * K, K)],
                        wsa1).wait()
  pltpu.make_async_copy(b1, pb_hbm.at[pl.ds(base0 + (CHW - 1) * K, K)],
                        wsb1).wait()


def _sc_pair_gather(a2, b2, row2, col2):
  kfn = pl.kernel(
      _pair_body,
      out_type=[jax.ShapeDtypeStruct((E, H), F32),
                jax.ShapeDtypeStruct((E, H), F32)],
      mesh=_mesh,
      compiler_params=_SC_PARAMS,
      scratch_types=[
          pltpu.VMEM((CHW, K), jnp.int32),
          pltpu.VMEM((CHW, K), jnp.int32),
          pltpu.VMEM((K, H), F32),
          pltpu.VMEM((K, H), F32),
          pltpu.VMEM((K, H), F32),
          pltpu.VMEM((K, H), F32),
          pltpu.SemaphoreType.DMA,
          pltpu.SemaphoreType.DMA,
          pltpu.SemaphoreType.DMA,
          pltpu.SemaphoreType.DMA,
          pltpu.SemaphoreType.DMA,
          pltpu.SemaphoreType.DMA,
          pltpu.SemaphoreType.DMA,
          pltpu.SemaphoreType.DMA,
      ],
  )
  return kfn(a2, b2, row2, col2)


def _t0_body(x_ref, w_ref, b_ref, h_ref):
  wbig = _bd4(w_ref[...])
  h_ref[...] = jnp.maximum(
      jnp.dot(x_ref[...], wbig, preferred_element_type=F32)
      + _tile4(b_ref[...]), 0.0)


def _tc_node_mlp(x_p, node_w, node_b):
  return pl.pallas_call(
      _t0_body,
      out_shape=jax.ShapeDtypeStruct((NP, 128), F32),
  )(x_p, node_w, node_b)


def _t1_body(degp_ref, h_ref, iw_ref, el1_ref, w1_ref, el2_ref,
             el1b_ref, el2b_ref, b1_ref,
             dinv_ref, g_ref, m1_ref, m2_ref, wea_ref, cvec_ref):
  deg = degp_ref[0] + degp_ref[1]
  safe = jnp.where(deg > 0, deg, 1.0)
  dinv32 = jnp.where(deg > 0, 1.0 / jnp.sqrt(safe), 0.0)
  li = lax.broadcasted_iota(jnp.int32, (32, 128), 0)
  mi = lax.broadcasted_iota(jnp.int32, (32, 128), 1)
  rep = jnp.where(li == 8 * (mi // H), 1.0, 0.0).astype(F32)
  dinv = jnp.dot(dinv32, rep, preferred_element_type=F32)
  dinv_ref[...] = dinv
  g_ref[...] = dinv * jnp.dot(h_ref[...], _bd4(iw_ref[...]),
                              preferred_element_type=F32)
  w1a = w1_ref[...][:H]
  w1b = w1_ref[...][H:]
  m1_ref[...] = jnp.dot(el1_ref[...][:H], w1a, preferred_element_type=F32)
  m2_ref[...] = jnp.dot(el1_ref[...][H:], w1a, preferred_element_type=F32)
  wea_ref[...] = jnp.dot(el2_ref[...], w1b, preferred_element_type=F32)
  cvec_ref[...] = (jnp.dot(el1b_ref[...], w1a, preferred_element_type=F32)
                   + jnp.dot(el2b_ref[...], w1b, preferred_element_type=F32)
                   + b1_ref[...])


def _tc_norm_pre(degp, h_p, init_w, el1_w, mlp_w1, el2_w,
                 el1_b, el2_b, mlp_b1):
  return pl.pallas_call(
      _t1_body,
      out_shape=[jax.ShapeDtypeStruct((NP, 128), F32),
                 jax.ShapeDtypeStruct((NP, 128), F32),
                 jax.ShapeDtypeStruct((H, H), F32),
                 jax.ShapeDtypeStruct((H, H), F32),
                 jax.ShapeDtypeStruct((16, H), F32),
                 jax.ShapeDtypeStruct((1, H), F32)],
  )(degp, h_p, init_w, el1_w, mlp_w1, el2_w, el1_b, el2_b, mlp_b1)


def _fold4(v):
  return (v[:, 0:32] + v[:, 32:64] + v[:, 64:96] + v[:, 96:128]) * 0.25


def _bn_post(aggp_ref, dinv_ref, h_ref, rw_ref, b_ref, g_ref, bt_ref):
  agg = aggp_ref[0] + aggp_ref[1]
  out = (dinv_ref[...] * agg
         + jnp.dot(h_ref[...], _bd4(rw_ref[...]), preferred_element_type=F32)
         + _tile4(b_ref[...]))
  out = jnp.maximum(out, 0.0)
  mean = _tile4(_fold4(jnp.mean(out, axis=0, keepdims=True)))
  d = out - mean
  var = _tile4(_fold4(jnp.mean(d * d, axis=0, keepdims=True)))
  return d / jnp.sqrt(var + 1e-5) * _tile4(g_ref[...]) + _tile4(bt_ref[...])


def _t2_body(aggp_ref, dinv_ref, h_ref, rw_ref, b_ref, g_ref, bt_ref, iw_ref,
             hout_ref, gout_ref):
  hn = _bn_post(aggp_ref, dinv_ref, h_ref, rw_ref, b_ref, g_ref, bt_ref)
  hout_ref[...] = hn
  gout_ref[...] = dinv_ref[...] * jnp.dot(hn, _bd4(iw_ref[...]),
                                          preferred_element_type=F32)


def _tc_post_pre(aggp, dinv, h, root_w, bias, gamma, beta, next_iw):
  return pl.pallas_call(
      _t2_body,
      out_shape=[jax.ShapeDtypeStruct((NP, 128), F32),
                 jax.ShapeDtypeStruct((NP, 128), F32)],
  )(aggp, dinv, h, root_w, bias, gamma, beta, next_iw)


def _t4_body(aggp_ref, dinv_ref, h_ref, rw_ref, b_ref, g_ref, bt_ref,
             m1_ref, m2_ref, a2_ref, b2_ref):
  hn = _bn_post(aggp_ref, dinv_ref, h_ref, rw_ref, b_ref, g_ref, bt_ref)
  a2_ref[...] = jnp.dot(hn, _bd4(m1_ref[...]), preferred_element_type=F32)
  b2_ref[...] = jnp.dot(hn, _bd4(m2_ref[...]), preferred_element_type=F32)


def _tc_final_post(aggp, dinv, h, root_w, bias, gamma, beta, m1, m2):
  return pl.pallas_call(
      _t4_body,
      out_shape=[jax.ShapeDtypeStruct((NP, 128), F32),
                 jax.ShapeDtypeStruct((NP, 128), F32)],
  )(aggp, dinv, h, root_w, bias, gamma, beta, m1, m2)


def _t5_body(pa_ref, pb_ref, ea_ref, wea_ref, cvec_ref, w2_ref, b2_ref,
             o_ref):
  wea = wea_ref[...]
  z16 = jnp.zeros((16, H), F32)
  rows = [jnp.concatenate([z16] * i + [wea] + [z16] * (3 - i), axis=1)
          for i in range(4)]
  wbig = jnp.concatenate(rows, axis=0)
  cvec4 = jnp.concatenate([cvec_ref[...]] * 4, axis=1)
  w2t = jnp.concatenate([w2_ref[...]] * 4, axis=1)
  li = lax.broadcasted_iota(jnp.int32, (128, 4), 0) // H
  ci = lax.broadcasted_iota(jnp.int32, (128, 4), 1)
  sel = jnp.where(li == ci, 1.0, 0.0).astype(F32)
  z = (pa_ref[...] + pb_ref[...]
       + jnp.dot(ea_ref[...], wbig, preferred_element_type=F32)
       + cvec4)
  zw = jnp.tanh(z) * w2t
  o_ref[...] = jnp.dot(zw, sel, preferred_element_type=F32) + b2_ref[...]


def _tc_edge_tail(pa4, pb4, ea4, wea, cvec, w2r, b2):
  eb4 = EB // 4
  nb = E // EB
  return pl.pallas_call(
      _t5_body,
      grid=(nb,),
      in_specs=[
          pl.BlockSpec((eb4, 128), lambda i: (i, 0)),
          pl.BlockSpec((eb4, 128), lambda i: (i, 0)),
          pl.BlockSpec((eb4, 64), lambda i: (i, 0)),
          pl.BlockSpec((16, H), lambda i: (0, 0)),
          pl.BlockSpec((1, H), lambda i: (0, 0)),
          pl.BlockSpec((1, H), lambda i: (0, 0)),
          pl.BlockSpec((1, 1), lambda i: (0, 0)),
      ],
      out_specs=pl.BlockSpec((eb4, 4), lambda i: (i, 0)),
      out_shape=jax.ShapeDtypeStruct((E // 4, 4), F32),
  )(pa4, pb4, ea4, wea, cvec, w2r, b2)


def kernel(x, edge_index, edge_attr, node_w, node_b,
           c0_init_w, c0_root_w, c0_bias, bn0_gamma, bn0_beta,
           c1_init_w, c1_root_w, c1_bias, bn1_gamma, bn1_beta,
           c2_init_w, c2_root_w, c2_bias, bn2_gamma, bn2_beta,
           el1_w, el1_b, el2_w, el2_b, mlp_w1, mlp_b1, mlp_w2, mlp_b2):
  row2 = edge_index[0].reshape(NW, CHW, K)
  col2 = edge_index[1].reshape(NW, CHW, K)
  ones_k = jnp.ones((K, DEGW), F32)
  zeros_stripe = jnp.zeros((640, H), F32)
  zeros_deg = jnp.zeros((640, DEGW), F32)

  degp = _sc_degree(col2, ones_k, zeros_deg)
  h = _tc_node_mlp(x.reshape(NP, 512), node_w, node_b.reshape(1, H))
  dinv, g, m1, m2, wea, cvec = _tc_norm_pre(
      degp.reshape(NC, NP, 32), h, c0_init_w[0],
      el1_w, mlp_w1, el2_w,
      el1_b.reshape(1, H), el2_b.reshape(1, H), mlp_b1.reshape(1, H))

  layers = [(c0_root_w, c0_bias, bn0_gamma, bn0_beta),
            (c1_root_w, c1_bias, bn1_gamma, bn1_beta),
            (c2_root_w, c2_bias, bn2_gamma, bn2_beta)]
  next_iws = [c1_init_w, c2_init_w]

  for ell in range(2):
    rw, b, gm, bt = layers[ell]
    aggp = _sc_gather_scatter(g.reshape(N, H), row2, col2, zeros_stripe)
    h, g = _tc_post_pre(aggp.reshape(NC, NP, 128), dinv, h, rw[0],
                        b.reshape(1, H), gm.reshape(1, H), bt.reshape(1, H),
                        next_iws[ell][0])

  rw, b, gm, bt = layers[2]
  aggp = _sc_gather_scatter(g.reshape(N, H), row2, col2, zeros_stripe)
  a2, b2 = _tc_final_post(
      aggp.reshape(NC, NP, 128), dinv, h, rw[0], b.reshape(1, H),
      gm.reshape(1, H), bt.reshape(1, H), m1, m2)

  pa, pb = _sc_pair_gather(a2.reshape(N, H), b2.reshape(N, H), row2, col2)

  out4 = _tc_edge_tail(pa.reshape(E // 4, 128), pb.reshape(E // 4, 128),
                       edge_attr.reshape(E // 4, 64), wea, cvec,
                       mlp_w2.reshape(1, H), mlp_b2.reshape(1, 1))
  return out4.reshape(-1)
